# baseline (speedup 1.0000x reference)
import functools

import jax
import jax.numpy as jnp
from jax import lax
from jax.experimental import pallas as pl
from jax.experimental.pallas import tpu as pltpu
from jax.experimental.pallas import tpu_sc as plsc

_B = 1024
_D = 32
_P = 5
_N = 10
_NROWS = _B * (1 + _P + _N)
_HALF = _NROWS // 2
_PAD = 128
_REM_BASE = 999936
_SUB = 8

_JB = 1024
_NBLK = (_P + _N) * _B // _JB
_POS_BLKS = _P * _B // _JB
_A_BLKS = _HALF // _JB - 1


def _gather_padded(idx, table_t, rem_t):
    nrows = idx.shape[0]
    info = plsc.get_sparse_core_info()
    nc, ns = info.num_cores, info.num_subcores
    nw = nc * ns
    kpw = nrows // nw
    nsub = kpw // _SUB
    mesh = plsc.VectorSubcoreMesh(core_axis_name="c", subcore_axis_name="s")

    @functools.partial(
        pl.kernel,
        mesh=mesh,
        out_type=jax.ShapeDtypeStruct((nrows, _PAD), jnp.float32),
        scratch_types=[
            pltpu.VMEM((kpw + 16,), jnp.int32),
            pltpu.VMEM((2, _SUB, _D, 128), jnp.float32),
            pltpu.VMEM((2, _SUB, _PAD), jnp.float32),
            pltpu.VMEM((_D, 128), jnp.float32),
            pltpu.SemaphoreType.DMA,
            pltpu.SemaphoreType.DMA,
            pltpu.SemaphoreType.DMA,
        ],
        compiler_params=pltpu.CompilerParams(needs_layout_passes=False),
    )
    def gather_k(idx_hbm, table_hbm, rem_hbm, out_hbm, idx_v, tiles_v,
                 rows_v, rem_v, fsem, osem0, osem1):
        wid = lax.axis_index("s") * nc + lax.axis_index("c")
        base = wid * kpw
        pltpu.sync_copy(idx_hbm.at[pl.ds(base, kpw)], idx_v.at[pl.ds(0, kpw)])
        pltpu.sync_copy(rem_hbm, rem_v)
        lanes = jnp.arange(16, dtype=jnp.int32)

        def issue(g, nb):
            vec = idx_v[pl.ds(g * _SUB, 16)]
            for u in range(_SUB):
                r = vec[u]
                q = jnp.minimum(r >> 7, jnp.int32(7811))
                pltpu.async_copy(
                    table_hbm.at[:, pl.ds(q * 128, 128)],
                    tiles_v.at[nb, u],
                    fsem,
                )

        def wait_fetches(nb):
            for u in range(_SUB):
                pltpu.make_async_copy(
                    table_hbm.at[:, pl.ds(0, 128)], tiles_v.at[nb, u], fsem
                ).wait()

        def extract(g, nb):
            vec = idx_v[pl.ds(g * _SUB, 16)]
            for u in range(_SUB):
                r = vec[u]
                q = jnp.minimum(r >> 7, jnp.int32(7811))
                l_main = jnp.minimum(r - q * 128, jnp.int32(127))
                l_rem = jnp.minimum(
                    jnp.maximum(r - _REM_BASE, jnp.int32(0)), jnp.int32(127)
                )
                in_rem = jnp.full((16,), r >= _REM_BASE, jnp.bool_)
                for h in range(2):
                    c = lanes + 16 * h
                    v_main = plsc.load_gather(
                        tiles_v,
                        [jnp.full((16,), nb, jnp.int32),
                         jnp.full((16,), u, jnp.int32),
                         c,
                         jnp.full((16,), l_main, jnp.int32)],
                    )
                    v_rem = plsc.load_gather(
                        rem_v, [c, jnp.full((16,), l_rem, jnp.int32)]
                    )
                    rows_v[nb, u, pl.ds(16 * h, 16)] = jnp.where(
                        in_rem, v_rem, v_main
                    )

        def ship(g, nb, osem):
            pltpu.async_copy(
                rows_v.at[nb],
                out_hbm.at[pl.ds(base + g * _SUB, _SUB)],
                osem,
            )

        def wait_ship(nb, osem):
            pltpu.make_async_copy(
                out_hbm.at[pl.ds(0, _SUB)], rows_v.at[nb], osem
            ).wait()

        issue(0, 0)

        def step(t, carry):
            g0 = 2 * t
            wait_fetches(0)
            issue(g0 + 1, 1)

            @pl.when(t > 0)
            def _():
                wait_ship(0, osem0)

            extract(g0, 0)
            ship(g0, 0, osem0)
            wait_fetches(1)

            @pl.when(t + 1 < nsub // 2)
            def _():
                issue(g0 + 2, 0)

            @pl.when(t > 0)
            def _():
                wait_ship(1, osem1)

            extract(g0 + 1, 1)
            ship(g0 + 1, 1, osem1)
            return carry

        lax.fori_loop(0, nsub // 2, step, 0)
        wait_ship(0, osem0)
        wait_ship(1, osem1)

    return gather_k(idx, table_t, rem_t)


def _make_cross(off, aliased):
    def body(*refs):
        center_ref, ctx_ref, out_ref = refs[0], refs[1], refs[-1]
        j = pl.program_id(0)
        sign = jnp.where(off + j < _POS_BLKS, jnp.float32(1.0),
                         jnp.float32(-1.0))
        acc = lax.dot_general(
            center_ref[:, :_D],
            ctx_ref[:, :_D],
            (((1,), (1,)), ((), ())),
            preferred_element_type=jnp.float32,
        )
        out_ref[...] = jax.nn.sigmoid(acc * sign)

    return body


def kernel(x_center, x_positive, x_negative, emb_table):
    idx = jnp.concatenate(
        [
            x_center.astype(jnp.int32).reshape(-1),
            x_positive.astype(jnp.int32).T.reshape(-1),
            x_negative.astype(jnp.int32).T.reshape(-1),
        ]
    )
    rem_t = jnp.concatenate(
        [
            emb_table[_REM_BASE:, :].T,
            jnp.zeros((_D, 128 - (emb_table.shape[0] - _REM_BASE)),
                      jnp.float32),
        ],
        axis=1,
    )
    table_t = emb_table.T
    quarter = _NROWS // 4
    pieces = [
        _gather_padded(idx[q * quarter:(q + 1) * quarter], table_t, rem_t)
        for q in range(4)
    ]
    out_shape = jax.ShapeDtypeStruct((_B, (_P + _N) * _B), jnp.float32)
    qb = quarter // _JB
    part = None
    for q in range(4):
        off = q * qb - 1
        nblk = qb if q else qb - 1
        in_specs = [
            pl.BlockSpec((_B, _PAD), lambda j: (0, 0)),
            pl.BlockSpec((_JB, _PAD),
                         (lambda j: (1 + j, 0)) if q == 0 else
                         (lambda j: (j, 0))),
        ]
        args = [pieces[0], pieces[q]]
        kwargs = {}
        if part is not None:
            in_specs.append(pl.BlockSpec(memory_space=pl.ANY))
            args.append(part)
            kwargs["input_output_aliases"] = {2: 0}
        part = pl.pallas_call(
            _make_cross(max(off, 0), part is not None),
            grid=(nblk,),
            in_specs=in_specs,
            out_specs=pl.BlockSpec(
                (_B, _JB),
                functools.partial(lambda j, o: (0, o + j), o=max(off, 0))),
            out_shape=out_shape,
            **kwargs,
        )(*args)
    return part


# speedup vs baseline: 1.0743x; 1.0743x over previous
import functools

import jax
import jax.numpy as jnp
from jax import lax
from jax.experimental import pallas as pl
from jax.experimental.pallas import tpu as pltpu
from jax.experimental.pallas import tpu_sc as plsc

_B = 1024
_D = 32
_P = 5
_N = 10
_NROWS = _B * (1 + _P + _N)
_PAD = 128
_REM_BASE = 999936
_SUB = 8

_JB = 1024
_NBLK = (_P + _N) * _B // _JB
_POS_BLKS = _P * _B // _JB


def _gather_padded(idx, table_t, rem_t):
    info = plsc.get_sparse_core_info()
    nc, ns = info.num_cores, info.num_subcores
    nw = nc * ns
    kpw = _NROWS // nw
    nsub = kpw // _SUB
    mesh = plsc.VectorSubcoreMesh(core_axis_name="c", subcore_axis_name="s")

    @functools.partial(
        pl.kernel,
        mesh=mesh,
        out_type=jax.ShapeDtypeStruct((_NROWS, _PAD), jnp.float32),
        scratch_types=[
            pltpu.VMEM((kpw + 16,), jnp.int32),
            pltpu.VMEM((2, _SUB, _D, 128), jnp.float32),
            pltpu.VMEM((2, _SUB, _PAD), jnp.float32),
            pltpu.VMEM((_D, 128), jnp.float32),
            pltpu.SemaphoreType.DMA,
            pltpu.SemaphoreType.DMA,
            pltpu.SemaphoreType.DMA,
        ],
        compiler_params=pltpu.CompilerParams(needs_layout_passes=False),
    )
    def gather_k(idx_hbm, table_hbm, rem_hbm, out_hbm, idx_v, tiles_v,
                 rows_v, rem_v, fsem, osem0, osem1):
        wid = lax.axis_index("s") * nc + lax.axis_index("c")
        base = wid * kpw
        pltpu.sync_copy(idx_hbm.at[pl.ds(base, kpw)], idx_v.at[pl.ds(0, kpw)])
        pltpu.sync_copy(rem_hbm, rem_v)
        lanes = jnp.arange(16, dtype=jnp.int32)

        def issue(g, nb):
            vec = idx_v[pl.ds(g * _SUB, 16)]
            for u in range(_SUB):
                r = vec[u]
                q = jnp.minimum(r >> 7, jnp.int32(7811))
                pltpu.async_copy(
                    table_hbm.at[:, pl.ds(q * 128, 128)],
                    tiles_v.at[nb, u],
                    fsem,
                )

        def wait_fetches(nb):
            for u in range(_SUB):
                pltpu.make_async_copy(
                    table_hbm.at[:, pl.ds(0, 128)], tiles_v.at[nb, u], fsem
                ).wait()

        def extract(g, nb):
            vec = idx_v[pl.ds(g * _SUB, 16)]
            for u in range(_SUB):
                r = vec[u]
                q = jnp.minimum(r >> 7, jnp.int32(7811))
                l_main = jnp.minimum(r - q * 128, jnp.int32(127))
                l_rem = jnp.minimum(
                    jnp.maximum(r - _REM_BASE, jnp.int32(0)), jnp.int32(127)
                )
                in_rem = jnp.full((16,), r >= _REM_BASE, jnp.bool_)
                for h in range(2):
                    c = lanes + 16 * h
                    v_main = plsc.load_gather(
                        tiles_v,
                        [jnp.full((16,), nb, jnp.int32),
                         jnp.full((16,), u, jnp.int32),
                         c,
                         jnp.full((16,), l_main, jnp.int32)],
                    )
                    v_rem = plsc.load_gather(
                        rem_v, [c, jnp.full((16,), l_rem, jnp.int32)]
                    )
                    rows_v[nb, u, pl.ds(16 * h, 16)] = jnp.where(
                        in_rem, v_rem, v_main
                    )

        def ship(g, nb, osem):
            pltpu.async_copy(
                rows_v.at[nb],
                out_hbm.at[pl.ds(base + g * _SUB, _SUB)],
                osem,
            )

        def wait_ship(nb, osem):
            pltpu.make_async_copy(
                out_hbm.at[pl.ds(0, _SUB)], rows_v.at[nb], osem
            ).wait()

        issue(0, 0)

        def step(t, carry):
            g0 = 2 * t
            wait_fetches(0)
            issue(g0 + 1, 1)

            @pl.when(t > 0)
            def _():
                wait_ship(0, osem0)

            extract(g0, 0)
            ship(g0, 0, osem0)
            wait_fetches(1)

            @pl.when(t + 1 < nsub // 2)
            def _():
                issue(g0 + 2, 0)

            @pl.when(t > 0)
            def _():
                wait_ship(1, osem1)

            extract(g0 + 1, 1)
            ship(g0 + 1, 1, osem1)
            return carry

        lax.fori_loop(0, nsub // 2, step, 0)
        wait_ship(0, osem0)
        wait_ship(1, osem1)

    return gather_k(idx, table_t, rem_t)


def _cross_body(center_ref, ctx_ref, out_ref):
    j = pl.program_id(0)
    sign = jnp.where(j < _POS_BLKS, jnp.float32(1.0), jnp.float32(-1.0))
    acc = lax.dot_general(
        center_ref[:, :_D],
        ctx_ref[:, :_D],
        (((1,), (1,)), ((), ())),
        preferred_element_type=jnp.float32,
    )
    out_ref[...] = jax.nn.sigmoid(acc * sign)


def kernel(x_center, x_positive, x_negative, emb_table):
    idx = jnp.concatenate(
        [
            x_center.astype(jnp.int32).reshape(-1),
            x_positive.astype(jnp.int32).T.reshape(-1),
            x_negative.astype(jnp.int32).T.reshape(-1),
        ]
    )
    rem_t = jnp.concatenate(
        [
            emb_table[_REM_BASE:, :].T,
            jnp.zeros((_D, 128 - (emb_table.shape[0] - _REM_BASE)),
                      jnp.float32),
        ],
        axis=1,
    )
    rows = _gather_padded(idx, emb_table.T, rem_t)
    return pl.pallas_call(
        _cross_body,
        grid=(_NBLK,),
        in_specs=[
            pl.BlockSpec((_B, _PAD), lambda j: (0, 0)),
            pl.BlockSpec((_JB, _PAD), lambda j: (1 + j, 0)),
        ],
        out_specs=pl.BlockSpec((_B, _JB), lambda j: (0, j)),
        out_shape=jax.ShapeDtypeStruct((_B, (_P + _N) * _B), jnp.float32),
    )(rows, rows)


# speedup vs baseline: 1.0799x; 1.0052x over previous
import functools

import jax
import jax.numpy as jnp
from jax import lax
from jax.experimental import pallas as pl
from jax.experimental.pallas import tpu as pltpu
from jax.experimental.pallas import tpu_sc as plsc

_B = 1024
_D = 32
_P = 5
_N = 10
_NROWS = _B * (1 + _P + _N)
_PAD = 128
_REM_BASE = 999936
_SUB = 8

_JB = 1024
_NBLK = (_P + _N) * _B // _JB
_POS_BLKS = _P * _B // _JB


def _gather_padded(idx, table_t, rem_t):
    nrows = idx.shape[0]
    info = plsc.get_sparse_core_info()
    nc, ns = info.num_cores, info.num_subcores
    nw = nc * ns
    kpw = nrows // nw
    nsub = kpw // _SUB
    mesh = plsc.VectorSubcoreMesh(core_axis_name="c", subcore_axis_name="s")

    @functools.partial(
        pl.kernel,
        mesh=mesh,
        out_type=jax.ShapeDtypeStruct((nrows, _PAD), jnp.float32),
        scratch_types=[
            pltpu.VMEM((kpw + 16,), jnp.int32),
            pltpu.VMEM((2, _SUB, _D, 128), jnp.float32),
            pltpu.VMEM((2, _SUB, _PAD), jnp.float32),
            pltpu.VMEM((_D, 128), jnp.float32),
            pltpu.SemaphoreType.DMA,
            pltpu.SemaphoreType.DMA,
            pltpu.SemaphoreType.DMA,
        ],
        compiler_params=pltpu.CompilerParams(needs_layout_passes=False),
    )
    def gather_k(idx_hbm, table_hbm, rem_hbm, out_hbm, idx_v, tiles_v,
                 rows_v, rem_v, fsem, osem0, osem1):
        wid = lax.axis_index("s") * nc + lax.axis_index("c")
        base = wid * kpw
        pltpu.sync_copy(idx_hbm.at[pl.ds(base, kpw)], idx_v.at[pl.ds(0, kpw)])
        pltpu.sync_copy(rem_hbm, rem_v)
        lanes = jnp.arange(16, dtype=jnp.int32)

        def issue(g, nb):
            vec = idx_v[pl.ds(g * _SUB, 16)]
            for u in range(_SUB):
                r = vec[u]
                q = jnp.minimum(r >> 7, jnp.int32(7811))
                pltpu.async_copy(
                    table_hbm.at[:, pl.ds(q * 128, 128)],
                    tiles_v.at[nb, u],
                    fsem,
                )

        def wait_fetches(nb):
            for u in range(_SUB):
                pltpu.make_async_copy(
                    table_hbm.at[:, pl.ds(0, 128)], tiles_v.at[nb, u], fsem
                ).wait()

        def extract(g, nb):
            vec = idx_v[pl.ds(g * _SUB, 16)]
            for u in range(_SUB):
                r = vec[u]
                q = jnp.minimum(r >> 7, jnp.int32(7811))
                l_main = jnp.minimum(r - q * 128, jnp.int32(127))
                l_rem = jnp.minimum(
                    jnp.maximum(r - _REM_BASE, jnp.int32(0)), jnp.int32(127)
                )
                in_rem = jnp.full((16,), r >= _REM_BASE, jnp.bool_)
                for h in range(2):
                    c = lanes + 16 * h
                    v_main = plsc.load_gather(
                        tiles_v,
                        [jnp.full((16,), nb, jnp.int32),
                         jnp.full((16,), u, jnp.int32),
                         c,
                         jnp.full((16,), l_main, jnp.int32)],
                    )
                    v_rem = plsc.load_gather(
                        rem_v, [c, jnp.full((16,), l_rem, jnp.int32)]
                    )
                    rows_v[nb, u, pl.ds(16 * h, 16)] = jnp.where(
                        in_rem, v_rem, v_main
                    )

        def ship(g, nb, osem):
            pltpu.async_copy(
                rows_v.at[nb],
                out_hbm.at[pl.ds(base + g * _SUB, _SUB)],
                osem,
            )

        def wait_ship(nb, osem):
            pltpu.make_async_copy(
                out_hbm.at[pl.ds(0, _SUB)], rows_v.at[nb], osem
            ).wait()

        issue(0, 0)

        def step(t, carry):
            g0 = 2 * t
            wait_fetches(0)
            issue(g0 + 1, 1)

            @pl.when(t > 0)
            def _():
                wait_ship(0, osem0)

            extract(g0, 0)
            ship(g0, 0, osem0)
            wait_fetches(1)

            @pl.when(t + 1 < nsub // 2)
            def _():
                issue(g0 + 2, 0)

            @pl.when(t > 0)
            def _():
                wait_ship(1, osem1)

            extract(g0 + 1, 1)
            ship(g0 + 1, 1, osem1)
            return carry

        lax.fori_loop(0, nsub // 2, step, 0)
        wait_ship(0, osem0)
        wait_ship(1, osem1)

    return gather_k(idx, table_t, rem_t)


def _cross_a(center_ref, ctx_ref, out_ref):
    j = pl.program_id(0)
    sign = jnp.where(j < _POS_BLKS, jnp.float32(1.0), jnp.float32(-1.0))
    acc = lax.dot_general(
        center_ref[:, :_D],
        ctx_ref[:, :_D],
        (((1,), (1,)), ((), ())),
        preferred_element_type=jnp.float32,
    )
    out_ref[...] = jax.nn.sigmoid(acc * sign)


def _cross_b(center_ref, ctx_ref, part_ref, out_ref):
    acc = lax.dot_general(
        center_ref[:, :_D],
        ctx_ref[:, :_D],
        (((1,), (1,)), ((), ())),
        preferred_element_type=jnp.float32,
    )
    out_ref[...] = jax.nn.sigmoid(-acc)


def kernel(x_center, x_positive, x_negative, emb_table):
    idx = jnp.concatenate(
        [
            x_center.astype(jnp.int32).reshape(-1),
            x_positive.astype(jnp.int32).T.reshape(-1),
            x_negative.astype(jnp.int32).T.reshape(-1),
        ]
    )
    rem_t = jnp.concatenate(
        [
            emb_table[_REM_BASE:, :].T,
            jnp.zeros((_D, 128 - (emb_table.shape[0] - _REM_BASE)),
                      jnp.float32),
        ],
        axis=1,
    )
    a_rows = (1 + 12) * _B
    rows_a = _gather_padded(idx[:a_rows], emb_table.T, rem_t)
    rows_b = _gather_padded(idx[a_rows:], emb_table.T, rem_t)
    out_shape = jax.ShapeDtypeStruct((_B, (_P + _N) * _B), jnp.float32)
    part = pl.pallas_call(
        _cross_a,
        grid=(12,),
        in_specs=[
            pl.BlockSpec((_B, _PAD), lambda j: (0, 0)),
            pl.BlockSpec((_JB, _PAD), lambda j: (1 + j, 0)),
        ],
        out_specs=pl.BlockSpec((_B, _JB), lambda j: (0, j)),
        out_shape=out_shape,
    )(rows_a, rows_a)
    return pl.pallas_call(
        _cross_b,
        grid=(3,),
        in_specs=[
            pl.BlockSpec((_B, _PAD), lambda j: (0, 0)),
            pl.BlockSpec((_JB, _PAD), lambda j: (j, 0)),
            pl.BlockSpec(memory_space=pl.ANY),
        ],
        out_specs=pl.BlockSpec((_B, _JB), lambda j: (0, 12 + j)),
        out_shape=out_shape,
        input_output_aliases={2: 0},
    )(rows_a, rows_b, part)
